# Initial kernel scaffold; baseline (speedup 1.0000x reference)
import jax
import jax.numpy as jnp
from jax.experimental import pallas as pl


def kernel(x, edge_index, W1, a_src1, a_dst1, b1, W2, a_src2, a_dst2, b2, fc_w, fc_b):
    raise NotImplementedError("write your pallas kernel here")


# speedup vs baseline: 38.6214x; 38.6214x over previous
import functools

import jax
import jax.numpy as jnp
from jax import lax
from jax.experimental import pallas as pl
from jax.experimental.pallas import tpu as pltpu
from jax.experimental.pallas import tpu_sc as plsc

N = 10000
D = 128
HID = 16
H1 = 8

NP = 10112
DUMMY = N
E = 320000
ET = E + N
K = 128
NTILES = 32
CHUNKS_PER_TILE = 81
PER_TILE = CHUNKS_PER_TILE * K
EP = NTILES * PER_TILE
NR = NP // 16


def _tc_call(f, out_shapes):
    return pl.pallas_call(f, out_shape=out_shapes)


def _stage_a(x_ref, w1_ref, as_ref, ad_ref, h_out, s_out, d_out):
    h = jnp.dot(x_ref[...], w1_ref[...], preferred_element_type=jnp.float32)
    h_out[...] = h
    s_out[...] = jnp.dot(h, as_ref[...], preferred_element_type=jnp.float32)
    d_out[...] = jnp.dot(h, ad_ref[...], preferred_element_type=jnp.float32)


def _stage_c(acc_ref, b1_ref, w2_ref, as2_ref, ad2_ref, r_ref,
             h2_out, s2_out, d2_out):
    a0 = acc_ref[0]
    a1 = acc_ref[1]
    num = a0[:, :128] + a1[:, :128]
    sw = a0[:, 128:] + a1[:, 128:]
    sden = jnp.dot(sw, r_ref[...], preferred_element_type=jnp.float32)
    h1 = num / sden + b1_ref[...]
    h1 = jnp.where(h1 > 0, h1, jnp.exp(h1) - 1.0)
    h2 = jnp.dot(h1, w2_ref[...], preferred_element_type=jnp.float32)
    h2_out[...] = h2
    s2_out[...] = jnp.dot(h2, as2_ref[...], preferred_element_type=jnp.float32)
    d2_out[...] = jnp.dot(h2, ad2_ref[...], preferred_element_type=jnp.float32)


def _stage_e(acc_ref, b2_ref, fcw_ref, fcb_ref, out_ref):
    a0 = acc_ref[0]
    a1 = acc_ref[1]
    num = a0[:, :16] + a1[:, :16]
    s = a0[:, 16:17] + a1[:, 16:17]
    h = num / s + b2_ref[...]
    h = jnp.where(h > 0, h, jnp.exp(h) - 1.0)
    y = jnp.dot(h, fcw_ref[...], preferred_element_type=jnp.float32)
    y = y + fcb_ref[...]
    m = jnp.max(y, axis=0, keepdims=True)
    p = jnp.exp(y - m)
    out_ref[...] = p / jnp.sum(p, axis=0, keepdims=True)


def _make_edge_pass(heads, hw):
    accw = hw + 16
    mesh = plsc.VectorSubcoreMesh(core_axis_name="c", subcore_axis_name="s")

    @functools.partial(
        pl.kernel,
        mesh=mesh,
        compiler_params=pltpu.CompilerParams(needs_layout_passes=False,
                                             use_tc_tiling_on_sc=False),
        out_type=jax.ShapeDtypeStruct((2, NP, accw), jnp.float32),
        scratch_types=[
            pltpu.VMEM_SHARED((NP, accw), jnp.float32),
            pltpu.VMEM((K,), jnp.int32),
            pltpu.VMEM((K,), jnp.int32),
            pltpu.VMEM((K, 16), jnp.float32),
            pltpu.VMEM((K, 16), jnp.float32),
            pltpu.VMEM((K, hw), jnp.float32),
            pltpu.VMEM((K, accw), jnp.float32),
            pltpu.VMEM((16,), jnp.float32),
            pltpu.SemaphoreType.DMA,
            pltpu.SemaphoreType.DMA,
            pltpu.SemaphoreType.DMA,
        ],
    )
    def edge_pass(src_h, dst_h, atts_h, attd_h, feat_h, z_h, out_h,
                  acc, srcv, dstv, asb, adb, hb, cb, wv, sem1, sem2, sem3):
        c = lax.axis_index("c")
        s = lax.axis_index("s")
        tile = c * 16 + s
        rows0 = s * NR
        pltpu.sync_copy(z_h.at[pl.ds(rows0, NR)], acc.at[pl.ds(rows0, NR)])
        plsc.subcore_barrier()
        base = tile * PER_TILE

        def chunk(i, carry):
            off = base + i * K
            pltpu.sync_copy(src_h.at[pl.ds(off, K)], srcv)
            pltpu.sync_copy(dst_h.at[pl.ds(off, K)], dstv)
            ga = pltpu.async_copy(atts_h.at[srcv], asb, sem1)
            gd = pltpu.async_copy(attd_h.at[dstv], adb, sem2)
            gh = pltpu.async_copy(feat_h.at[srcv], hb, sem3)
            ga.wait()
            gd.wait()
            gh.wait()

            def edge(e, carry2):
                ev = asb[e] + adb[e]
                ev = jnp.where(ev >= 0.0, ev, 0.2 * ev)
                w = jnp.exp(ev)
                cb[e, pl.ds(hw, 16)] = w
                wv[...] = w
                for hh in range(heads):
                    idx = jnp.full((16,), hh, jnp.int32)
                    wb = plsc.load_gather(wv, [idx])
                    cb[e, pl.ds(hh * 16, 16)] = wb * hb[e, pl.ds(hh * 16, 16)]
                return carry2

            lax.fori_loop(0, K, edge, 0)
            pltpu.sync_copy(cb, acc.at[dstv], add=True)
            return carry

        lax.fori_loop(0, CHUNKS_PER_TILE, chunk, 0)
        plsc.subcore_barrier()
        pltpu.sync_copy(acc.at[pl.ds(rows0, NR)],
                        out_h.at[c, pl.ds(rows0, NR)])

    return edge_pass


_edge_pass_l1 = _make_edge_pass(H1, H1 * HID)
_edge_pass_l2 = _make_edge_pass(1, HID)


def kernel(x, edge_index, W1, a_src1, a_dst1, b1, W2, a_src2, a_dst2, b2,
           fc_w, fc_b):
    f32 = jnp.float32
    xp = jnp.zeros((NP, D), f32).at[:N].set(x)
    loop = jnp.arange(N, dtype=jnp.int32)
    pad = jnp.full((EP - ET,), DUMMY, jnp.int32)
    src = jnp.concatenate([edge_index[0], loop, pad])
    dst = jnp.concatenate([edge_index[1], loop, pad])

    eye = jnp.eye(H1, 16, dtype=f32)
    As1 = (a_src1[:, :, None] * eye[:, None, :]).reshape(D, 16)
    Ad1 = (a_dst1[:, :, None] * eye[:, None, :]).reshape(D, 16)
    As2 = jnp.pad(a_src2.T, ((0, 0), (0, 15)))
    Ad2 = jnp.pad(a_dst2.T, ((0, 0), (0, 15)))
    R = (jnp.arange(128)[None, :] // 16 ==
         jnp.arange(16)[:, None]).astype(f32)
    z1 = jnp.zeros((NP, 144), f32)
    z2 = jnp.zeros((NP, 32), f32)

    sh = jax.ShapeDtypeStruct
    h1, as1, ad1 = _tc_call(_stage_a, [sh((NP, D), f32), sh((NP, 16), f32),
                                       sh((NP, 16), f32)])(xp, W1, As1, Ad1)
    acc1 = _edge_pass_l1(src, dst, as1, ad1, h1, z1)
    h2, as2v, ad2v = _tc_call(
        _stage_c, [sh((NP, 16), f32), sh((NP, 16), f32), sh((NP, 16), f32)])(
            acc1, b1.reshape(1, D), W2, As2, Ad2, R)
    acc2 = _edge_pass_l2(src, dst, as2v, ad2v, h2, z2)
    out = _tc_call(_stage_e, sh((N, 1), f32))(
        acc2[:, :N], b2.reshape(1, HID), fc_w, fc_b.reshape(1, 1))
    return out


# speedup vs baseline: 45.8660x; 1.1876x over previous
import functools

import jax
import jax.numpy as jnp
from jax import lax
from jax.experimental import pallas as pl
from jax.experimental.pallas import tpu as pltpu
from jax.experimental.pallas import tpu_sc as plsc

N = 10000
D = 128
HID = 16
H1 = 8

NP = 10112
DUMMY = N
E = 320000
ET = E + N
K = 96
NTILES = 32
CH = 112
PER_TILE = CH * K
EP = NTILES * PER_TILE
NR = NP // 16


def _tc_call(f, out_shapes):
    return pl.pallas_call(f, out_shape=out_shapes)


def _stage_a(x_ref, w1_ref, as_ref, ad_ref, hx_out, d_out):
    h = jnp.dot(x_ref[...], w1_ref[...], preferred_element_type=jnp.float32)
    hx_out[:, :D] = h
    hx_out[:, D:] = jnp.dot(h, as_ref[...], preferred_element_type=jnp.float32)
    d_out[...] = jnp.dot(h, ad_ref[...], preferred_element_type=jnp.float32)


def _stage_c(acc_ref, b1_ref, w2_ref, as2_ref, ad2_ref, r_ref,
             hx2_out, d2_out):
    a0 = acc_ref[0]
    a1 = acc_ref[1]
    num = a0[:, :128] + a1[:, :128]
    sw = a0[:, 128:] + a1[:, 128:]
    sden = jnp.dot(sw, r_ref[...], preferred_element_type=jnp.float32)
    h1 = num / sden + b1_ref[...]
    h1 = jnp.where(h1 > 0, h1, jnp.exp(h1) - 1.0)
    h2 = jnp.dot(h1, w2_ref[...], preferred_element_type=jnp.float32)
    hx2_out[:, :16] = h2
    hx2_out[:, 16:] = jnp.dot(h2, as2_ref[...],
                              preferred_element_type=jnp.float32)
    d2_out[...] = jnp.dot(h2, ad2_ref[...], preferred_element_type=jnp.float32)


def _stage_e(acc_ref, b2_ref, fcw_ref, fcb_ref, out_ref):
    a0 = acc_ref[0]
    a1 = acc_ref[1]
    num = a0[:, :16] + a1[:, :16]
    s = a0[:, 16:17] + a1[:, 16:17]
    h = num / s + b2_ref[...]
    h = jnp.where(h > 0, h, jnp.exp(h) - 1.0)
    y = jnp.dot(h, fcw_ref[...], preferred_element_type=jnp.float32)
    y = y + fcb_ref[...]
    m = jnp.max(y, axis=0, keepdims=True)
    p = jnp.exp(y - m)
    out_ref[...] = p / jnp.sum(p, axis=0, keepdims=True)


def _make_edge_pass(heads, hw):
    accw = hw + 16
    mesh = plsc.VectorSubcoreMesh(core_axis_name="c", subcore_axis_name="s")

    @functools.partial(
        pl.kernel,
        mesh=mesh,
        compiler_params=pltpu.CompilerParams(needs_layout_passes=False,
                                             use_tc_tiling_on_sc=False),
        out_type=jax.ShapeDtypeStruct((2, NP, accw), jnp.float32),
        scratch_types=[
            pltpu.VMEM_SHARED((NP, accw), jnp.float32),
            pltpu.VMEM((2, K), jnp.int32),
            pltpu.VMEM((2, K), jnp.int32),
            pltpu.VMEM((2, K), jnp.int32),
            pltpu.VMEM((2, K), jnp.int32),
            pltpu.VMEM((K, accw), jnp.float32),
            pltpu.VMEM((K, accw), jnp.float32),
            pltpu.VMEM((K, 16), jnp.float32),
            pltpu.VMEM((K, 16), jnp.float32),
            pltpu.SemaphoreType.DMA,
            pltpu.SemaphoreType.DMA,
            pltpu.SemaphoreType.DMA,
            pltpu.SemaphoreType.DMA,
            pltpu.SemaphoreType.DMA,
            pltpu.SemaphoreType.DMA,
        ],
    )
    def edge_pass(sd_h, hx_h, attd_h, z_h, out_h,
                  acc, sd0, sd1, sd2, sd3, hb0, hb1, ab0, ab1,
                  gs0, gs1, ss0, ss1, is0, is1):
        c = lax.axis_index("c")
        s = lax.axis_index("s")
        tile = c * 16 + s
        rows0 = s * NR
        cbase = tile * CH
        pltpu.sync_copy(z_h.at[pl.ds(rows0, NR)], acc.at[pl.ds(rows0, NR)])

        sds = (sd0, sd1, sd2, sd3)
        hbs = (hb0, hb1)
        abs_ = (ab0, ab1)
        gss = (gs0, gs1)
        sss = (ss0, ss1)
        iss = (is0, is1)

        def issue_gather(r, bb):
            pltpu.async_copy(hx_h.at[sds[r].at[0]], hbs[bb], gss[bb])
            pltpu.async_copy(attd_h.at[sds[r].at[1]], abs_[bb], gss[bb])

        pltpu.sync_copy(sd_h.at[cbase], sd0)
        pltpu.sync_copy(sd_h.at[cbase + 1], sd1)
        plsc.subcore_barrier()
        issue_gather(0, 0)

        def quad(g, carry):
            for u in range(4):
                b = u % 2
                cur = 4 * g + u
                hbx, abx, gsx, ssx = hbs[b], abs_[b], gss[b], sss[b]
                pltpu.make_async_copy(hx_h.at[pl.ds(0, K)], hbx, gsx).wait()
                pltpu.make_async_copy(attd_h.at[pl.ds(0, K)], abx,
                                      gsx).wait()

                @pl.when(cur >= 1)
                def _():
                    pltpu.make_async_copy(hbs[1 - b], acc.at[pl.ds(0, K)],
                                          sss[1 - b]).wait()

                @pl.when(cur + 1 < CH)
                def _():
                    @pl.when(cur >= 1)
                    def _():
                        pltpu.make_async_copy(sd_h.at[cbase],
                                              sds[(u + 1) % 4],
                                              iss[1 - b]).wait()
                    issue_gather((u + 1) % 4, 1 - b)

                @pl.when(cur + 2 < CH)
                def _():
                    pltpu.async_copy(sd_h.at[cbase + cur + 2],
                                     sds[(u + 2) % 4], iss[b])

                @plsc.parallel_loop(0, K, unroll=2)
                def edge(e):
                    ad = abx[e]
                    a_s = hbx[e, pl.ds(hw, 16)]
                    ev = a_s + ad
                    ev = jnp.where(ev >= 0.0, ev, 0.2 * ev)
                    w = jnp.exp(ev)
                    hbx[e, pl.ds(hw, 16)] = w
                    for hh in range(heads):
                        idx = jnp.full((16,), hh, jnp.int32)
                        wb = w.at[idx].get(mode="promise_in_bounds")
                        hbx[e, pl.ds(hh * 16, 16)] = (
                            wb * hbx[e, pl.ds(hh * 16, 16)])

                pltpu.async_copy(hbx, acc.at[sds[u].at[1]], ssx, add=True)
            return carry

        lax.fori_loop(0, CH // 4, quad, 0)
        pltpu.make_async_copy(hbs[(CH - 1) % 2], acc.at[pl.ds(0, K)],
                              sss[(CH - 1) % 2]).wait()
        plsc.subcore_barrier()
        pltpu.sync_copy(acc.at[pl.ds(rows0, NR)],
                        out_h.at[c, pl.ds(rows0, NR)])

    return edge_pass


_edge_pass_l1 = _make_edge_pass(H1, H1 * HID)
_edge_pass_l2 = _make_edge_pass(1, HID)


def kernel(x, edge_index, W1, a_src1, a_dst1, b1, W2, a_src2, a_dst2, b2,
           fc_w, fc_b):
    f32 = jnp.float32
    xp = jnp.zeros((NP, D), f32).at[:N].set(x)
    loop = jnp.arange(N, dtype=jnp.int32)
    pad = jnp.full((EP - ET,), DUMMY, jnp.int32)
    src = jnp.concatenate([edge_index[0], loop, pad]).reshape(NTILES * CH, K)
    dst = jnp.concatenate([edge_index[1], loop, pad]).reshape(NTILES * CH, K)
    sd = jnp.stack([src, dst], axis=1)

    eye = jnp.eye(H1, 16, dtype=f32)
    As1 = (a_src1[:, :, None] * eye[:, None, :]).reshape(D, 16)
    Ad1 = (a_dst1[:, :, None] * eye[:, None, :]).reshape(D, 16)
    As2 = jnp.pad(a_src2.T, ((0, 0), (0, 15)))
    Ad2 = jnp.pad(a_dst2.T, ((0, 0), (0, 15)))
    R = (jnp.arange(128)[None, :] // 16 ==
         jnp.arange(16)[:, None]).astype(f32)
    z1 = jnp.zeros((NP, 144), f32)
    z2 = jnp.zeros((NP, 32), f32)

    sh = jax.ShapeDtypeStruct
    hx1, ad1 = _tc_call(_stage_a, [sh((NP, 144), f32), sh((NP, 16), f32)])(
        xp, W1, As1, Ad1)
    acc1 = _edge_pass_l1(sd, hx1, ad1, z1)
    hx2, ad2v = _tc_call(
        _stage_c, [sh((NP, 32), f32), sh((NP, 16), f32)])(
            acc1, b1.reshape(1, D), W2, As2, Ad2, R)
    acc2 = _edge_pass_l2(sd, hx2, ad2v, z2)
    out = _tc_call(_stage_e, sh((N, 1), f32))(
        acc2[:, :N], b2.reshape(1, HID), fc_w, fc_b.reshape(1, 1))
    return out


# speedup vs baseline: 128.2944x; 2.7972x over previous
import functools

import jax
import jax.numpy as jnp
from jax import lax
from jax.experimental import pallas as pl
from jax.experimental.pallas import tpu as pltpu
from jax.experimental.pallas import tpu_sc as plsc

N = 10000
D = 128
HID = 16
H1 = 8

NP = 10112
DUMMY = N
E = 320000
ET = E + N
K = 112
NTILES = 32
CH = 96
PER_TILE = CH * K
EP = NTILES * PER_TILE
NR = NP // 16


def _tc_call(f, out_shapes):
    return pl.pallas_call(f, out_shape=out_shapes)


def _stage_a(x_ref, w1_ref, as_ref, ad_ref, hx_out, d_out):
    h = jnp.dot(x_ref[...], w1_ref[...], preferred_element_type=jnp.float32)
    hx_out[:, :D] = h
    hx_out[:, D:] = jnp.dot(h, as_ref[...], preferred_element_type=jnp.float32)
    d_out[...] = jnp.dot(h, ad_ref[...], preferred_element_type=jnp.float32)


def _stage_c(acc_ref, b1_ref, w2_ref, as2_ref, ad2_ref, r_ref,
             hx2_out, d2_out):
    a0 = acc_ref[0]
    a1 = acc_ref[1]
    num = a0[:, :128] + a1[:, :128]
    sw = a0[:, 128:] + a1[:, 128:]
    sden = jnp.dot(sw, r_ref[...], preferred_element_type=jnp.float32)
    h1 = num / sden + b1_ref[...]
    h1 = jnp.where(h1 > 0, h1, jnp.exp(h1) - 1.0)
    h2 = jnp.dot(h1, w2_ref[...], preferred_element_type=jnp.float32)
    hx2_out[:, :16] = h2
    hx2_out[:, 16:] = jnp.dot(h2, as2_ref[...],
                              preferred_element_type=jnp.float32)
    d2_out[...] = jnp.dot(h2, ad2_ref[...], preferred_element_type=jnp.float32)


def _stage_e(acc_ref, b2_ref, fcw_ref, fcb_ref, out_ref):
    a0 = acc_ref[0]
    a1 = acc_ref[1]
    num = a0[:, :16] + a1[:, :16]
    s = a0[:, 16:17] + a1[:, 16:17]
    h = num / s + b2_ref[...]
    h = jnp.where(h > 0, h, jnp.exp(h) - 1.0)
    y = jnp.dot(h, fcw_ref[...], preferred_element_type=jnp.float32)
    y = y + fcb_ref[...]
    m = jnp.max(y, axis=0, keepdims=True)
    p = jnp.exp(y - m)
    out_ref[...] = p / jnp.sum(p, axis=0, keepdims=True)


def _make_edge_pass(heads, hw):
    accw = hw + 16
    mesh = plsc.VectorSubcoreMesh(core_axis_name="c", subcore_axis_name="s")

    @functools.partial(
        pl.kernel,
        mesh=mesh,
        compiler_params=pltpu.CompilerParams(needs_layout_passes=False,
                                             use_tc_tiling_on_sc=False),
        out_type=jax.ShapeDtypeStruct((2, NP, accw), jnp.float32),
        scratch_types=[
            pltpu.VMEM_SHARED((NP, accw), jnp.float32),
            pltpu.VMEM((2, K), jnp.int32),
            pltpu.VMEM((2, K), jnp.int32),
            pltpu.VMEM((2, K), jnp.int32),
            pltpu.VMEM((2, K), jnp.int32),
            pltpu.VMEM((K, accw), jnp.float32),
            pltpu.VMEM((K, accw), jnp.float32),
            pltpu.VMEM((K, 16), jnp.float32),
            pltpu.VMEM((K, 16), jnp.float32),
            pltpu.SemaphoreType.DMA,
            pltpu.SemaphoreType.DMA,
            pltpu.SemaphoreType.DMA,
            pltpu.SemaphoreType.DMA,
            pltpu.SemaphoreType.DMA,
            pltpu.SemaphoreType.DMA,
        ],
    )
    def edge_pass(sd_h, hx_h, attd_h, z_h, out_h,
                  acc, sd0, sd1, sd2, sd3, hb0, hb1, ab0, ab1,
                  gs0, gs1, ss0, ss1, is0, is1):
        c = lax.axis_index("c")
        s = lax.axis_index("s")
        tile = c * 16 + s
        rows0 = s * NR
        cbase = tile * CH
        pltpu.sync_copy(z_h.at[pl.ds(rows0, NR)], acc.at[pl.ds(rows0, NR)])

        sds = (sd0, sd1, sd2, sd3)
        hbs = (hb0, hb1)
        abs_ = (ab0, ab1)
        gss = (gs0, gs1)
        sss = (ss0, ss1)
        iss = (is0, is1)

        def issue_gather(r, bb):
            pltpu.async_copy(hx_h.at[sds[r].at[0]], hbs[bb], gss[bb])
            pltpu.async_copy(attd_h.at[sds[r].at[1]], abs_[bb], gss[bb])

        pltpu.sync_copy(sd_h.at[cbase], sd0)
        pltpu.sync_copy(sd_h.at[cbase + 1], sd1)
        plsc.subcore_barrier()
        issue_gather(0, 0)

        def quad(g, carry):
            for u in range(4):
                b = u % 2
                cur = 4 * g + u
                hbx, abx, gsx, ssx = hbs[b], abs_[b], gss[b], sss[b]
                pltpu.make_async_copy(hx_h.at[pl.ds(0, K)], hbx, gsx).wait()
                pltpu.make_async_copy(attd_h.at[pl.ds(0, K)], abx,
                                      gsx).wait()

                @pl.when(cur >= 1)
                def _():
                    pltpu.make_async_copy(hbs[1 - b], acc.at[pl.ds(0, K)],
                                          sss[1 - b]).wait()

                @pl.when(cur + 1 < CH)
                def _():
                    @pl.when(cur >= 1)
                    def _():
                        pltpu.make_async_copy(sd_h.at[cbase],
                                              sds[(u + 1) % 4],
                                              iss[1 - b]).wait()
                    issue_gather((u + 1) % 4, 1 - b)

                @pl.when(cur + 2 < CH)
                def _():
                    pltpu.async_copy(sd_h.at[cbase + cur + 2],
                                     sds[(u + 2) % 4], iss[b])

                @plsc.parallel_loop(0, K, unroll=2)
                def edge(e):
                    ad = abx[e]
                    a_s = hbx[e, pl.ds(hw, 16)]
                    ev = a_s + ad
                    ev = jnp.where(ev >= 0.0, ev, 0.2 * ev)
                    w = jnp.exp(ev)
                    hbx[e, pl.ds(hw, 16)] = w
                    for hh in range(heads):
                        idx = jnp.full((16,), hh, jnp.int32)
                        wb = w.at[idx].get(mode="promise_in_bounds")
                        hbx[e, pl.ds(hh * 16, 16)] = (
                            wb * hbx[e, pl.ds(hh * 16, 16)])

                pltpu.async_copy(hbx, acc.at[sds[u].at[1]], ssx, add=True)
            return carry

        lax.fori_loop(0, CH // 4, quad, 0)
        pltpu.make_async_copy(hbs[(CH - 1) % 2], acc.at[pl.ds(0, K)],
                              sss[(CH - 1) % 2]).wait()
        plsc.subcore_barrier()
        pltpu.sync_copy(acc.at[pl.ds(rows0, NR)],
                        out_h.at[c, pl.ds(rows0, NR)])

    return edge_pass


_edge_pass_l1 = _make_edge_pass(H1, H1 * HID)
_edge_pass_l2 = _make_edge_pass(1, HID)


def kernel(x, edge_index, W1, a_src1, a_dst1, b1, W2, a_src2, a_dst2, b2,
           fc_w, fc_b):
    f32 = jnp.float32
    xp = jnp.zeros((NP, D), f32).at[:N].set(x)
    loop = jnp.arange(N, dtype=jnp.int32)
    pad = DUMMY + (jnp.arange(EP - ET, dtype=jnp.int32) % (NP - N))
    src = jnp.concatenate([edge_index[0], loop, pad]).reshape(NTILES * CH, K)
    dst = jnp.concatenate([edge_index[1], loop, pad]).reshape(NTILES * CH, K)
    sd = jnp.stack([src, dst], axis=1)

    eye = jnp.eye(H1, 16, dtype=f32)
    As1 = (a_src1[:, :, None] * eye[:, None, :]).reshape(D, 16)
    Ad1 = (a_dst1[:, :, None] * eye[:, None, :]).reshape(D, 16)
    As2 = jnp.pad(a_src2.T, ((0, 0), (0, 15)))
    Ad2 = jnp.pad(a_dst2.T, ((0, 0), (0, 15)))
    R = (jnp.arange(128)[None, :] // 16 ==
         jnp.arange(16)[:, None]).astype(f32)
    z1 = jnp.zeros((NP, 144), f32)
    z2 = jnp.zeros((NP, 32), f32)

    sh = jax.ShapeDtypeStruct
    hx1, ad1 = _tc_call(_stage_a, [sh((NP, 144), f32), sh((NP, 16), f32)])(
        xp, W1, As1, Ad1)
    acc1 = _edge_pass_l1(sd, hx1, ad1, z1)
    hx2, ad2v = _tc_call(
        _stage_c, [sh((NP, 32), f32), sh((NP, 16), f32)])(
            acc1, b1.reshape(1, D), W2, As2, Ad2, R)
    acc2 = _edge_pass_l2(sd, hx2, ad2v, z2)
    out = _tc_call(_stage_e, sh((N, 1), f32))(
        acc2[:, :N], b2.reshape(1, HID), fc_w, fc_b.reshape(1, 1))
    return out


# speedup vs baseline: 129.7890x; 1.0116x over previous
import functools

import jax
import jax.numpy as jnp
from jax import lax
from jax.experimental import pallas as pl
from jax.experimental.pallas import tpu as pltpu
from jax.experimental.pallas import tpu_sc as plsc

N = 10000
D = 128
HID = 16
H1 = 8

NP = 10112
DUMMY = N
E = 320000
ET = E + N
NTILES = 32
PER_TILE = 10752
EP = NTILES * PER_TILE
NR = NP // 16


def _tc_call(f, out_shapes):
    return pl.pallas_call(f, out_shape=out_shapes)


def _stage_a(x_ref, w1_ref, as_ref, ad_ref, hx_out, d_out):
    h = jnp.dot(x_ref[...], w1_ref[...], preferred_element_type=jnp.float32)
    hx_out[:, :D] = h
    hx_out[:, D:] = jnp.dot(h, as_ref[...], preferred_element_type=jnp.float32)
    d_out[...] = jnp.dot(h, ad_ref[...], preferred_element_type=jnp.float32)


def _stage_c(acc_ref, b1_ref, w2_ref, as2_ref, ad2_ref, r_ref,
             hx2_out, d2_out):
    a0 = acc_ref[0]
    a1 = acc_ref[1]
    num = a0[:, :128] + a1[:, :128]
    sw = a0[:, 128:] + a1[:, 128:]
    sden = jnp.dot(sw, r_ref[...], preferred_element_type=jnp.float32)
    h1 = num / sden + b1_ref[...]
    h1 = jnp.where(h1 > 0, h1, jnp.exp(h1) - 1.0)
    h2 = jnp.dot(h1, w2_ref[...], preferred_element_type=jnp.float32)
    hx2_out[:, :16] = h2
    hx2_out[:, 16:] = jnp.dot(h2, as2_ref[...],
                              preferred_element_type=jnp.float32)
    d2_out[...] = jnp.dot(h2, ad2_ref[...], preferred_element_type=jnp.float32)


def _stage_e(acc_ref, b2_ref, fcw_ref, fcb_ref, out_ref):
    a0 = acc_ref[0]
    a1 = acc_ref[1]
    num = a0[:, :16] + a1[:, :16]
    s = a0[:, 16:17] + a1[:, 16:17]
    h = num / s + b2_ref[...]
    h = jnp.where(h > 0, h, jnp.exp(h) - 1.0)
    y = jnp.dot(h, fcw_ref[...], preferred_element_type=jnp.float32)
    y = y + fcb_ref[...]
    m = jnp.max(y, axis=0, keepdims=True)
    p = jnp.exp(y - m)
    out_ref[...] = p / jnp.sum(p, axis=0, keepdims=True)


def _make_edge_pass(heads, hw, K, CH, unroll):
    assert K * CH == PER_TILE and CH % 4 == 0
    accw = hw + 16
    mesh = plsc.VectorSubcoreMesh(core_axis_name="c", subcore_axis_name="s")

    @functools.partial(
        pl.kernel,
        mesh=mesh,
        compiler_params=pltpu.CompilerParams(needs_layout_passes=False,
                                             use_tc_tiling_on_sc=False),
        out_type=jax.ShapeDtypeStruct((2, NP, accw), jnp.float32),
        scratch_types=[
            pltpu.VMEM_SHARED((NP, accw), jnp.float32),
            pltpu.VMEM((2, K), jnp.int32),
            pltpu.VMEM((2, K), jnp.int32),
            pltpu.VMEM((2, K), jnp.int32),
            pltpu.VMEM((2, K), jnp.int32),
            pltpu.VMEM((K, accw), jnp.float32),
            pltpu.VMEM((K, accw), jnp.float32),
            pltpu.VMEM((K, 16), jnp.float32),
            pltpu.VMEM((K, 16), jnp.float32),
            pltpu.SemaphoreType.DMA,
            pltpu.SemaphoreType.DMA,
            pltpu.SemaphoreType.DMA,
            pltpu.SemaphoreType.DMA,
            pltpu.SemaphoreType.DMA,
            pltpu.SemaphoreType.DMA,
        ],
    )
    def edge_pass(sd_h, hx_h, attd_h, z_h, out_h,
                  acc, sd0, sd1, sd2, sd3, hb0, hb1, ab0, ab1,
                  gs0, gs1, ss0, ss1, is0, is1):
        c = lax.axis_index("c")
        s = lax.axis_index("s")
        tile = c * 16 + s
        rows0 = s * NR
        cbase = tile * CH

        sds = (sd0, sd1, sd2, sd3)
        hbs = (hb0, hb1)
        abs_ = (ab0, ab1)
        gss = (gs0, gs1)
        sss = (ss0, ss1)
        iss = (is0, is1)

        def issue_gather(r, bb):
            pltpu.async_copy(hx_h.at[sds[r].at[0]], hbs[bb], gss[bb])
            pltpu.async_copy(attd_h.at[sds[r].at[1]], abs_[bb], gss[bb])

        pltpu.sync_copy(sd_h.at[cbase], sd0)
        pltpu.sync_copy(sd_h.at[cbase + 1], sd1)
        issue_gather(0, 0)
        pltpu.sync_copy(z_h.at[pl.ds(rows0, NR)], acc.at[pl.ds(rows0, NR)])
        plsc.subcore_barrier()

        def quad(g, carry):
            for u in range(4):
                b = u % 2
                cur = 4 * g + u
                hbx, abx, gsx, ssx = hbs[b], abs_[b], gss[b], sss[b]
                pltpu.make_async_copy(hx_h.at[pl.ds(0, K)], hbx, gsx).wait()
                pltpu.make_async_copy(attd_h.at[pl.ds(0, K)], abx,
                                      gsx).wait()

                @pl.when(cur >= 1)
                def _():
                    pltpu.make_async_copy(hbs[1 - b], acc.at[pl.ds(0, K)],
                                          sss[1 - b]).wait()

                @pl.when(cur + 1 < CH)
                def _():
                    @pl.when(cur >= 1)
                    def _():
                        pltpu.make_async_copy(sd_h.at[cbase],
                                              sds[(u + 1) % 4],
                                              iss[1 - b]).wait()
                    issue_gather((u + 1) % 4, 1 - b)

                @pl.when(cur + 2 < CH)
                def _():
                    pltpu.async_copy(sd_h.at[cbase + cur + 2],
                                     sds[(u + 2) % 4], iss[b])

                @plsc.parallel_loop(0, K, unroll=unroll)
                def edge(e):
                    ad = abx[e]
                    a_s = hbx[e, pl.ds(hw, 16)]
                    ev = a_s + ad
                    ev = jnp.where(ev >= 0.0, ev, 0.2 * ev)
                    w = jnp.exp(ev)
                    hbx[e, pl.ds(hw, 16)] = w
                    for hh in range(heads):
                        idx = jnp.full((16,), hh, jnp.int32)
                        wb = w.at[idx].get(mode="promise_in_bounds")
                        hbx[e, pl.ds(hh * 16, 16)] = (
                            wb * hbx[e, pl.ds(hh * 16, 16)])

                pltpu.async_copy(hbx, acc.at[sds[u].at[1]], ssx, add=True)
            return carry

        lax.fori_loop(0, CH // 4, quad, 0)
        pltpu.make_async_copy(hbs[(CH - 1) % 2], acc.at[pl.ds(0, K)],
                              sss[(CH - 1) % 2]).wait()
        plsc.subcore_barrier()
        pltpu.sync_copy(acc.at[pl.ds(rows0, NR)],
                        out_h.at[c, pl.ds(rows0, NR)])

    return edge_pass


K1, CH1 = 112, 96
K2, CH2 = 128, 84
_edge_pass_l1 = _make_edge_pass(H1, H1 * HID, K1, CH1, 2)
_edge_pass_l2 = _make_edge_pass(1, HID, K2, CH2, 4)


def kernel(x, edge_index, W1, a_src1, a_dst1, b1, W2, a_src2, a_dst2, b2,
           fc_w, fc_b):
    f32 = jnp.float32
    xp = jnp.zeros((NP, D), f32).at[:N].set(x)
    loop = jnp.arange(N, dtype=jnp.int32)
    pad = DUMMY + (jnp.arange(EP - ET, dtype=jnp.int32) % (NP - N))
    src = jnp.concatenate([edge_index[0], loop, pad])
    dst = jnp.concatenate([edge_index[1], loop, pad])
    sd1 = jnp.stack([src.reshape(NTILES * CH1, K1),
                     dst.reshape(NTILES * CH1, K1)], axis=1)
    sd2 = jnp.stack([src.reshape(NTILES * CH2, K2),
                     dst.reshape(NTILES * CH2, K2)], axis=1)

    eye = jnp.eye(H1, 16, dtype=f32)
    As1 = (a_src1[:, :, None] * eye[:, None, :]).reshape(D, 16)
    Ad1 = (a_dst1[:, :, None] * eye[:, None, :]).reshape(D, 16)
    As2 = jnp.pad(a_src2.T, ((0, 0), (0, 15)))
    Ad2 = jnp.pad(a_dst2.T, ((0, 0), (0, 15)))
    R = (jnp.arange(128)[None, :] // 16 ==
         jnp.arange(16)[:, None]).astype(f32)
    z1 = jnp.zeros((NP, 144), f32)
    z2 = jnp.zeros((NP, 32), f32)

    sh = jax.ShapeDtypeStruct
    hx1, ad1 = _tc_call(_stage_a, [sh((NP, 144), f32), sh((NP, 16), f32)])(
        xp, W1, As1, Ad1)
    acc1 = _edge_pass_l1(sd1, hx1, ad1, z1)
    hx2, ad2v = _tc_call(
        _stage_c, [sh((NP, 32), f32), sh((NP, 16), f32)])(
            acc1, b1.reshape(1, D), W2, As2, Ad2, R)
    acc2 = _edge_pass_l2(sd2, hx2, ad2v, z2)
    out = _tc_call(_stage_e, sh((N, 1), f32))(
        acc2[:, :N], b2.reshape(1, HID), fc_w, fc_b.reshape(1, 1))
    return out


# speedup vs baseline: 131.0491x; 1.0097x over previous
import functools

import jax
import jax.numpy as jnp
from jax import lax
from jax.experimental import pallas as pl
from jax.experimental.pallas import tpu as pltpu
from jax.experimental.pallas import tpu_sc as plsc

N = 10000
D = 128
HID = 16
H1 = 8

NP = 10112
DUMMY = N
E = 320000
ET = E + N
NTILES = 32
PER_TILE = 10752
EP = NTILES * PER_TILE
NR = NP // 16


def _tc_call(f, out_shapes):
    return pl.pallas_call(f, out_shape=out_shapes)


def _stage_a(x_ref, w1_ref, as_ref, ad_ref, hx_out, d_out):
    h = jnp.dot(x_ref[...], w1_ref[...], preferred_element_type=jnp.float32)
    hx_out[:, :D] = h
    hx_out[:, D:] = jnp.dot(h, as_ref[...], preferred_element_type=jnp.float32)
    d_out[...] = jnp.dot(h, ad_ref[...], preferred_element_type=jnp.float32)


def _stage_c(acc_ref, b1_ref, w2_ref, as2_ref, ad2_ref, r_ref,
             hx2_out, d2_out):
    a0 = acc_ref[0]
    a1 = acc_ref[1]
    num = a0[:, :128] + a1[:, :128]
    sw = a0[:, 128:] + a1[:, 128:]
    sden = jnp.dot(sw, r_ref[...], preferred_element_type=jnp.float32)
    h1 = num / sden + b1_ref[...]
    h1 = jnp.where(h1 > 0, h1, jnp.exp(h1) - 1.0)
    h2 = jnp.dot(h1, w2_ref[...], preferred_element_type=jnp.float32)
    hx2_out[:, :16] = h2
    hx2_out[:, 16:] = jnp.dot(h2, as2_ref[...],
                              preferred_element_type=jnp.float32)
    d2_out[...] = jnp.dot(h2, ad2_ref[...], preferred_element_type=jnp.float32)


def _stage_e(acc_ref, b2_ref, fcw_ref, fcb_ref, out_ref):
    a0 = acc_ref[0]
    a1 = acc_ref[1]
    num = a0[:, :16] + a1[:, :16]
    s = a0[:, 16:17] + a1[:, 16:17]
    h = num / s + b2_ref[...]
    h = jnp.where(h > 0, h, jnp.exp(h) - 1.0)
    y = jnp.dot(h, fcw_ref[...], preferred_element_type=jnp.float32)
    y = y + fcb_ref[...]
    m = jnp.max(y, axis=0, keepdims=True)
    p = jnp.exp(y - m)
    out_ref[...] = p / jnp.sum(p, axis=0, keepdims=True)


def _make_edge_pass(heads, hw, K, CH, unroll):
    assert K * CH == PER_TILE and CH % 4 == 0
    accw = hw + 16
    mesh = plsc.VectorSubcoreMesh(core_axis_name="c", subcore_axis_name="s")

    @functools.partial(
        pl.kernel,
        mesh=mesh,
        compiler_params=pltpu.CompilerParams(needs_layout_passes=False,
                                             use_tc_tiling_on_sc=False),
        out_type=jax.ShapeDtypeStruct((2, NP, accw), jnp.float32),
        scratch_types=[
            pltpu.VMEM_SHARED((NP, accw), jnp.float32),
            pltpu.VMEM((2, K), jnp.int32),
            pltpu.VMEM((2, K), jnp.int32),
            pltpu.VMEM((2, K), jnp.int32),
            pltpu.VMEM((2, K), jnp.int32),
            pltpu.VMEM((K, accw), jnp.float32),
            pltpu.VMEM((K, accw), jnp.float32),
            pltpu.VMEM((K, 16), jnp.float32),
            pltpu.VMEM((K, 16), jnp.float32),
            pltpu.SemaphoreType.DMA,
            pltpu.SemaphoreType.DMA,
            pltpu.SemaphoreType.DMA,
            pltpu.SemaphoreType.DMA,
            pltpu.SemaphoreType.DMA,
            pltpu.SemaphoreType.DMA,
        ],
    )
    def edge_pass(sd_h, hx_h, attd_h, out_h,
                  acc, sd0, sd1, sd2, sd3, hb0, hb1, ab0, ab1,
                  gs0, gs1, ss0, ss1, is0, is1):
        c = lax.axis_index("c")
        s = lax.axis_index("s")
        tile = c * 16 + s
        rows0 = s * NR
        cbase = tile * CH

        sds = (sd0, sd1, sd2, sd3)
        hbs = (hb0, hb1)
        abs_ = (ab0, ab1)
        gss = (gs0, gs1)
        sss = (ss0, ss1)
        iss = (is0, is1)

        def issue_gather(r, bb):
            pltpu.async_copy(hx_h.at[sds[r].at[0]], hbs[bb], gss[bb])
            pltpu.async_copy(attd_h.at[sds[r].at[1]], abs_[bb], gss[bb])

        pltpu.sync_copy(sd_h.at[cbase], sd0)
        pltpu.sync_copy(sd_h.at[cbase + 1], sd1)
        issue_gather(0, 0)

        @plsc.parallel_loop(0, K, unroll=2)
        def zrow(r):
            for j in range(accw // 16):
                hb1[r, pl.ds(j * 16, 16)] = jnp.zeros((16,), jnp.float32)

        for j in range(NR // K):
            pltpu.sync_copy(hb1, acc.at[pl.ds(rows0 + j * K, K)])
        if NR % K:
            pltpu.sync_copy(hb1.at[pl.ds(0, NR % K)],
                            acc.at[pl.ds(rows0 + (NR // K) * K, NR % K)])
        plsc.subcore_barrier()

        def quad(g, carry):
            for u in range(4):
                b = u % 2
                cur = 4 * g + u
                hbx, abx, gsx, ssx = hbs[b], abs_[b], gss[b], sss[b]
                pltpu.make_async_copy(hx_h.at[pl.ds(0, K)], hbx, gsx).wait()
                pltpu.make_async_copy(attd_h.at[pl.ds(0, K)], abx,
                                      gsx).wait()

                @pl.when(cur >= 1)
                def _():
                    pltpu.make_async_copy(hbs[1 - b], acc.at[pl.ds(0, K)],
                                          sss[1 - b]).wait()

                @pl.when(cur + 1 < CH)
                def _():
                    @pl.when(cur >= 1)
                    def _():
                        pltpu.make_async_copy(sd_h.at[cbase],
                                              sds[(u + 1) % 4],
                                              iss[1 - b]).wait()
                    issue_gather((u + 1) % 4, 1 - b)

                @pl.when(cur + 2 < CH)
                def _():
                    pltpu.async_copy(sd_h.at[cbase + cur + 2],
                                     sds[(u + 2) % 4], iss[b])

                @plsc.parallel_loop(0, K, unroll=unroll)
                def edge(e):
                    ad = abx[e]
                    a_s = hbx[e, pl.ds(hw, 16)]
                    ev = a_s + ad
                    ev = jnp.where(ev >= 0.0, ev, 0.2 * ev)
                    w = jnp.exp(ev)
                    hbx[e, pl.ds(hw, 16)] = w
                    for hh in range(heads):
                        idx = jnp.full((16,), hh, jnp.int32)
                        wb = w.at[idx].get(mode="promise_in_bounds")
                        hbx[e, pl.ds(hh * 16, 16)] = (
                            wb * hbx[e, pl.ds(hh * 16, 16)])

                pltpu.async_copy(hbx, acc.at[sds[u].at[1]], ssx, add=True)
            return carry

        lax.fori_loop(0, CH // 4, quad, 0)
        pltpu.make_async_copy(hbs[(CH - 1) % 2], acc.at[pl.ds(0, K)],
                              sss[(CH - 1) % 2]).wait()
        plsc.subcore_barrier()
        pltpu.sync_copy(acc.at[pl.ds(rows0, NR)],
                        out_h.at[c, pl.ds(rows0, NR)])

    return edge_pass


K1, CH1 = 112, 96
_edge_pass_l1 = _make_edge_pass(H1, H1 * HID, K1, CH1, 4)
_edge_pass_l2 = _make_edge_pass(1, HID, K1, CH1, 4)


def kernel(x, edge_index, W1, a_src1, a_dst1, b1, W2, a_src2, a_dst2, b2,
           fc_w, fc_b):
    f32 = jnp.float32
    xp = jnp.zeros((NP, D), f32).at[:N].set(x)
    loop = jnp.arange(N, dtype=jnp.int32)
    pad = DUMMY + (jnp.arange(EP - ET, dtype=jnp.int32) % (NP - N))
    src = jnp.concatenate([edge_index[0], loop, pad])
    dst = jnp.concatenate([edge_index[1], loop, pad])
    sd1 = jnp.stack([src.reshape(NTILES * CH1, K1),
                     dst.reshape(NTILES * CH1, K1)], axis=1)

    eye = jnp.eye(H1, 16, dtype=f32)
    As1 = (a_src1[:, :, None] * eye[:, None, :]).reshape(D, 16)
    Ad1 = (a_dst1[:, :, None] * eye[:, None, :]).reshape(D, 16)
    As2 = jnp.pad(a_src2.T, ((0, 0), (0, 15)))
    Ad2 = jnp.pad(a_dst2.T, ((0, 0), (0, 15)))
    R = (jnp.arange(128)[None, :] // 16 ==
         jnp.arange(16)[:, None]).astype(f32)
    sh = jax.ShapeDtypeStruct
    hx1, ad1 = _tc_call(_stage_a, [sh((NP, 144), f32), sh((NP, 16), f32)])(
        xp, W1, As1, Ad1)
    acc1 = _edge_pass_l1(sd1, hx1, ad1)
    hx2, ad2v = _tc_call(
        _stage_c, [sh((NP, 32), f32), sh((NP, 16), f32)])(
            acc1, b1.reshape(1, D), W2, As2, Ad2, R)
    acc2 = _edge_pass_l2(sd1, hx2, ad2v)
    out = _tc_call(_stage_e, sh((N, 1), f32))(
        acc2[:, :N], b2.reshape(1, HID), fc_w, fc_b.reshape(1, 1))
    return out


# speedup vs baseline: 135.4220x; 1.0334x over previous
import functools

import jax
import jax.numpy as jnp
from jax import lax
from jax.experimental import pallas as pl
from jax.experimental.pallas import tpu as pltpu
from jax.experimental.pallas import tpu_sc as plsc

N = 10000
D = 128
HID = 16
H1 = 8

NP = 10112
DUMMY = N
E = 320000
ET = E + N
NTILES = 32
PER_TILE = 10752
EP = NTILES * PER_TILE
NR = NP // 16
K, CH = 112, 96


def _tc_call(f, out_shapes):
    return pl.pallas_call(f, out_shape=out_shapes)


def _stage_a(x_ref, w1_ref, as_ref, ad_ref, hbf_out, s_out, d_out):
    h = jnp.dot(x_ref[...], w1_ref[...], preferred_element_type=jnp.float32)
    hbf_out[...] = h.astype(jnp.bfloat16)
    s_out[...] = jnp.dot(h, as_ref[...], preferred_element_type=jnp.float32)
    d_out[...] = jnp.dot(h, ad_ref[...], preferred_element_type=jnp.float32)


def _stage_c(acc_ref, b1_ref, rq_ref, m2_ref, nad_ref, hx2_out, d2_out):
    a0 = acc_ref[0]
    a1 = acc_ref[1]
    num = a0[:, :128] + a1[:, :128]
    sw = a0[:, 128:] + a1[:, 128:]
    sden = jnp.dot(sw, rq_ref[...], preferred_element_type=jnp.float32)
    h1 = num / sden + b1_ref[...]
    h1 = jnp.where(h1 > 0, h1, jnp.exp(h1) - 1.0)
    hx2_out[...] = jnp.dot(h1, m2_ref[...],
                           preferred_element_type=jnp.float32
                           ).astype(jnp.bfloat16)
    d2_out[...] = jnp.dot(h1, nad_ref[...], preferred_element_type=jnp.float32)


def _stage_e(acc_ref, b2_ref, fcw_ref, fcb_ref, out_ref):
    a0 = acc_ref[0]
    a1 = acc_ref[1]
    num = a0[:, :16] + a1[:, :16]
    s = a0[:, 16:17] + a1[:, 16:17]
    h = num / s + b2_ref[...]
    h = jnp.where(h > 0, h, jnp.exp(h) - 1.0)
    y = jnp.dot(h, fcw_ref[...], preferred_element_type=jnp.float32)
    y = y + fcb_ref[...]
    m = jnp.max(y, axis=0, keepdims=True)
    p = jnp.exp(y - m)
    out_ref[...] = p / jnp.sum(p, axis=0, keepdims=True)


def _make_edge_pass(heads, hw):
    accw = hw + 16
    sep_as = heads > 1
    hbw = 128 if sep_as else 32
    mesh = plsc.VectorSubcoreMesh(core_axis_name="c", subcore_axis_name="s")

    scratch = [
        pltpu.VMEM_SHARED((NP, accw), jnp.float32),
        pltpu.VMEM((2, K), jnp.int32),
        pltpu.VMEM((2, K), jnp.int32),
        pltpu.VMEM((2, K), jnp.int32),
        pltpu.VMEM((2, K), jnp.int32),
        pltpu.VMEM((K, hbw), jnp.bfloat16),
        pltpu.VMEM((K, hbw), jnp.bfloat16),
    ]
    if sep_as:
        scratch += [pltpu.VMEM((K, 16), jnp.float32),
                    pltpu.VMEM((K, 16), jnp.float32)]
    scratch += [
        pltpu.VMEM((K, 16), jnp.float32),
        pltpu.VMEM((K, 16), jnp.float32),
        pltpu.VMEM((K, accw), jnp.float32),
        pltpu.SemaphoreType.DMA,
        pltpu.SemaphoreType.DMA,
        pltpu.SemaphoreType.DMA,
        pltpu.SemaphoreType.DMA,
        pltpu.SemaphoreType.DMA,
        pltpu.SemaphoreType.DMA,
    ]

    @functools.partial(
        pl.kernel,
        mesh=mesh,
        compiler_params=pltpu.CompilerParams(needs_layout_passes=False,
                                             use_tc_tiling_on_sc=False),
        out_type=jax.ShapeDtypeStruct((2, NP, accw), jnp.float32),
        scratch_types=scratch,
    )
    def edge_pass(*refs):
        it = iter(refs)
        sd_h = next(it)
        hx_h = next(it)
        as_h = next(it) if sep_as else None
        ad_h = next(it)
        out_h = next(it)
        acc = next(it)
        sds = (next(it), next(it), next(it), next(it))
        hbs = (next(it), next(it))
        if sep_as:
            ass = (next(it), next(it))
        abs_ = (next(it), next(it))
        cb = next(it)
        gss = (next(it), next(it))
        sss = (next(it), next(it))
        iss = (next(it), next(it))

        c = lax.axis_index("c")
        s = lax.axis_index("s")
        tile = c * 16 + s
        rows0 = s * NR
        cbase = tile * CH

        def issue_gather(r, bb):
            pltpu.async_copy(hx_h.at[sds[r].at[0]], hbs[bb], gss[bb])
            if sep_as:
                pltpu.async_copy(as_h.at[sds[r].at[0]], ass[bb], gss[bb])
            pltpu.async_copy(ad_h.at[sds[r].at[1]], abs_[bb], gss[bb])

        def wait_gather(bb):
            pltpu.make_async_copy(hx_h.at[pl.ds(0, K)], hbs[bb],
                                  gss[bb]).wait()
            if sep_as:
                pltpu.make_async_copy(as_h.at[pl.ds(0, K)], ass[bb],
                                      gss[bb]).wait()
            pltpu.make_async_copy(ad_h.at[pl.ds(0, K)], abs_[bb],
                                  gss[bb]).wait()

        pltpu.sync_copy(sd_h.at[cbase], sds[0])
        pltpu.sync_copy(sd_h.at[cbase + 1], sds[1])
        issue_gather(0, 0)

        @plsc.parallel_loop(0, K, unroll=2)
        def zrow(r):
            for j in range(accw // 16):
                cb[r, pl.ds(j * 16, 16)] = jnp.zeros((16,), jnp.float32)

        for j in range(NR // K):
            pltpu.sync_copy(cb, acc.at[pl.ds(rows0 + j * K, K)])
        if NR % K:
            pltpu.sync_copy(cb.at[pl.ds(0, NR % K)],
                            acc.at[pl.ds(rows0 + (NR // K) * K, NR % K)])
        plsc.subcore_barrier()

        def quad(g, carry):
            for u in range(4):
                b = u % 2
                cur = 4 * g + u
                hbx = hbs[b]
                abx = abs_[b]

                wait_gather(b)

                @pl.when(cur >= 1)
                def _():
                    pltpu.make_async_copy(cb, acc.at[pl.ds(0, K)],
                                          sss[1 - b]).wait()

                @pl.when(cur + 1 < CH)
                def _():
                    @pl.when(cur >= 1)
                    def _():
                        pltpu.make_async_copy(sd_h.at[cbase],
                                              sds[(u + 1) % 4],
                                              iss[1 - b]).wait()
                    issue_gather((u + 1) % 4, 1 - b)

                @pl.when(cur + 2 < CH)
                def _():
                    pltpu.async_copy(sd_h.at[cbase + cur + 2],
                                     sds[(u + 2) % 4], iss[b])

                @plsc.parallel_loop(0, K, unroll=2)
                def edge(e):
                    ad = abx[e]
                    if sep_as:
                        a_s = ass[b][e]
                    else:
                        h2v, a_s = plsc.unpack(
                            hbx[e], format=plsc.PackFormat.INTERLEAVED)
                    ev = a_s + ad
                    ev = jnp.where(ev >= 0.0, ev, 0.2 * ev)
                    w = jnp.exp(ev)
                    cb[e, pl.ds(hw, 16)] = w
                    if sep_as:
                        for gg in range(4):
                            idx = jnp.int32(2 * gg) + (
                                jax.lax.iota(jnp.int32, 16) // 8)
                            wpair = w.at[idx].get(mode="promise_in_bounds")
                            ve, vo = plsc.unpack(
                                hbx[e, pl.ds(gg * 32, 32)],
                                format=plsc.PackFormat.INTERLEAVED)
                            cb[e, pl.ds(gg * 32, 16)] = ve * wpair
                            cb[e, pl.ds(gg * 32 + 16, 16)] = vo * wpair
                    else:
                        idx0 = jnp.zeros((16,), jnp.int32)
                        wb = w.at[idx0].get(mode="promise_in_bounds")
                        cb[e, pl.ds(0, 16)] = wb * h2v

                pltpu.async_copy(cb, acc.at[sds[u].at[1]], sss[b], add=True)
            return carry

        lax.fori_loop(0, CH // 4, quad, 0)
        pltpu.make_async_copy(cb, acc.at[pl.ds(0, K)],
                              sss[(CH - 1) % 2]).wait()
        plsc.subcore_barrier()
        pltpu.sync_copy(acc.at[pl.ds(rows0, NR)],
                        out_h.at[c, pl.ds(rows0, NR)])

    return edge_pass


_edge_pass_l1 = _make_edge_pass(H1, H1 * HID)
_edge_pass_l2 = _make_edge_pass(1, HID)


def kernel(x, edge_index, W1, a_src1, a_dst1, b1, W2, a_src2, a_dst2, b2,
           fc_w, fc_b):
    f32 = jnp.float32
    xp = jnp.zeros((NP, D), f32).at[:N].set(x)
    loop = jnp.arange(N, dtype=jnp.int32)
    pad = DUMMY + (jnp.arange(EP - ET, dtype=jnp.int32) % (NP - N))
    src = jnp.concatenate([edge_index[0], loop, pad])
    dst = jnp.concatenate([edge_index[1], loop, pad])
    sd = jnp.stack([src.reshape(NTILES * CH, K),
                    dst.reshape(NTILES * CH, K)], axis=1)

    cc = jnp.arange(D)
    m = (cc % 32) // 2
    P = 16 * (2 * (cc // 32) + m // 8) + 2 * (m % 8) + (cc % 2)
    Q = P[32 * (cc // 32) + 2 * (cc % 16) + (cc % 32) // 16]

    eye = jnp.eye(H1, 16, dtype=f32)
    As1 = (a_src1[:, :, None] * eye[:, None, :]).reshape(D, 16)
    Ad1 = (a_dst1[:, :, None] * eye[:, None, :]).reshape(D, 16)
    W1p = W1[:, P]
    As1p = As1[P]
    Ad1p = Ad1[P]
    RQ = (Q[None, :] // 16 == jnp.arange(16)[:, None]).astype(f32)
    b1Q = b1[Q].reshape(1, D)
    W2Q = W2[Q]
    av2 = W2Q @ a_src2[0]
    M2 = jnp.zeros((D, 32), f32).at[:, 0::2].set(W2Q).at[:, 1].set(av2)
    Ad2 = jnp.pad(a_dst2.T, ((0, 0), (0, 15)))
    NAd2 = W2Q @ Ad2

    sh = jax.ShapeDtypeStruct
    hbf, as1, ad1 = _tc_call(
        _stage_a, [sh((NP, D), jnp.bfloat16), sh((NP, 16), f32),
                   sh((NP, 16), f32)])(xp, W1p, As1p, Ad1p)
    acc1 = _edge_pass_l1(sd, hbf, as1, ad1)
    hx2, ad2v = _tc_call(
        _stage_c, [sh((NP, 32), jnp.bfloat16), sh((NP, 16), f32)])(
            acc1, b1Q, RQ, M2, NAd2)
    acc2 = _edge_pass_l2(sd, hx2, ad2v)
    out = _tc_call(_stage_e, sh((N, 1), f32))(
        acc2[:, :N], b2.reshape(1, HID), fc_w, fc_b.reshape(1, 1))
    return out


# speedup vs baseline: 135.7182x; 1.0022x over previous
import functools

import jax
import jax.numpy as jnp
from jax import lax
from jax.experimental import pallas as pl
from jax.experimental.pallas import tpu as pltpu
from jax.experimental.pallas import tpu_sc as plsc

N = 10000
D = 128
HID = 16
H1 = 8

NP = 10112
DUMMY = N
E = 320000
ET = E + N
NTILES = 32
PER_TILE = 10752
EP = NTILES * PER_TILE
NR = NP // 16
K, CH = 112, 96


def _tc_call(f, out_shapes):
    return pl.pallas_call(f, out_shape=out_shapes)


def _stage_a(x_ref, c1_ref, g1d_ref, hbf_out, d_out):
    x = x_ref[...]
    h = jnp.dot(x, c1_ref[...], preferred_element_type=jnp.float32)
    hbf_out[...] = h.astype(jnp.bfloat16)
    d_out[...] = jnp.dot(x, g1d_ref[...], preferred_element_type=jnp.float32)


def _stage_c(acc_ref, b1_ref, rq_ref, m2_ref, nad_ref, hx2_out, d2_out):
    a0 = acc_ref[0]
    a1 = acc_ref[1]
    num = a0[:, :128] + a1[:, :128]
    sw = a0[:, 128:] + a1[:, 128:]
    sden = jnp.dot(sw, rq_ref[...], preferred_element_type=jnp.float32)
    h1 = num / sden + b1_ref[...]
    h1 = jnp.where(h1 > 0, h1, jnp.exp(h1) - 1.0)
    hx2_out[...] = jnp.dot(h1, m2_ref[...],
                           preferred_element_type=jnp.float32
                           ).astype(jnp.bfloat16)
    d2_out[...] = jnp.dot(h1, nad_ref[...], preferred_element_type=jnp.float32)


def _stage_e(acc_ref, b2_ref, fcw_ref, fcb_ref, out_ref):
    a0 = acc_ref[0]
    a1 = acc_ref[1]
    num = a0[:, :16] + a1[:, :16]
    s = a0[:, 16:17] + a1[:, 16:17]
    h = num / s + b2_ref[...]
    h = jnp.where(h > 0, h, jnp.exp(h) - 1.0)
    y = jnp.dot(h, fcw_ref[...], preferred_element_type=jnp.float32)
    y = y + fcb_ref[...]
    m = jnp.max(y, axis=0, keepdims=True)
    p = jnp.exp(y - m)
    out_ref[...] = p / jnp.sum(p, axis=0, keepdims=True)


def _make_edge_pass(heads, hw):
    accw = hw + 16
    multi = heads > 1
    hbw = 160 if multi else 32
    mesh = plsc.VectorSubcoreMesh(core_axis_name="c", subcore_axis_name="s")

    scratch = [
        pltpu.VMEM_SHARED((NP, accw), jnp.float32),
        pltpu.VMEM((2, K), jnp.int32),
        pltpu.VMEM((2, K), jnp.int32),
        pltpu.VMEM((2, K), jnp.int32),
        pltpu.VMEM((2, K), jnp.int32),
        pltpu.VMEM((K, hbw), jnp.bfloat16),
        pltpu.VMEM((K, hbw), jnp.bfloat16),
        pltpu.VMEM((K, 16), jnp.float32),
        pltpu.VMEM((K, 16), jnp.float32),
        pltpu.VMEM((K, accw), jnp.float32),
        pltpu.SemaphoreType.DMA,
        pltpu.SemaphoreType.DMA,
        pltpu.SemaphoreType.DMA,
        pltpu.SemaphoreType.DMA,
        pltpu.SemaphoreType.DMA,
        pltpu.SemaphoreType.DMA,
    ]

    @functools.partial(
        pl.kernel,
        mesh=mesh,
        compiler_params=pltpu.CompilerParams(needs_layout_passes=False,
                                             use_tc_tiling_on_sc=False),
        out_type=jax.ShapeDtypeStruct((2, NP, accw), jnp.float32),
        scratch_types=scratch,
    )
    def edge_pass(*refs):
        it = iter(refs)
        sd_h = next(it)
        hx_h = next(it)
        ad_h = next(it)
        out_h = next(it)
        acc = next(it)
        sds = (next(it), next(it), next(it), next(it))
        hbs = (next(it), next(it))
        abs_ = (next(it), next(it))
        cb = next(it)
        gss = (next(it), next(it))
        sss = (next(it), next(it))
        iss = (next(it), next(it))

        c = lax.axis_index("c")
        s = lax.axis_index("s")
        tile = c * 16 + s
        rows0 = s * NR
        cbase = tile * CH

        def issue_gather(r, bb):
            pltpu.async_copy(hx_h.at[sds[r].at[0]], hbs[bb], gss[bb])
            pltpu.async_copy(ad_h.at[sds[r].at[1]], abs_[bb], gss[bb])

        def wait_gather(bb):
            pltpu.make_async_copy(hx_h.at[pl.ds(0, K)], hbs[bb],
                                  gss[bb]).wait()
            pltpu.make_async_copy(ad_h.at[pl.ds(0, K)], abs_[bb],
                                  gss[bb]).wait()

        pltpu.sync_copy(sd_h.at[cbase], sds[0])
        pltpu.sync_copy(sd_h.at[cbase + 1], sds[1])
        issue_gather(0, 0)

        @plsc.parallel_loop(0, K, unroll=2)
        def zrow(r):
            for j in range(accw // 16):
                cb[r, pl.ds(j * 16, 16)] = jnp.zeros((16,), jnp.float32)

        for j in range(NR // K):
            pltpu.sync_copy(cb, acc.at[pl.ds(rows0 + j * K, K)])
        if NR % K:
            pltpu.sync_copy(cb.at[pl.ds(0, NR % K)],
                            acc.at[pl.ds(rows0 + (NR // K) * K, NR % K)])
        plsc.subcore_barrier()

        def quad(g, carry):
            for u in range(4):
                b = u % 2
                cur = 4 * g + u
                hbx = hbs[b]
                abx = abs_[b]

                wait_gather(b)

                @pl.when(cur >= 1)
                def _():
                    pltpu.make_async_copy(cb, acc.at[pl.ds(0, K)],
                                          sss[1 - b]).wait()

                @pl.when(cur + 1 < CH)
                def _():
                    @pl.when(cur >= 1)
                    def _():
                        pltpu.make_async_copy(sd_h.at[cbase],
                                              sds[(u + 1) % 4],
                                              iss[1 - b]).wait()
                    issue_gather((u + 1) % 4, 1 - b)

                @pl.when(cur + 2 < CH)
                def _():
                    pltpu.async_copy(sd_h.at[cbase + cur + 2],
                                     sds[(u + 2) % 4], iss[b])

                @plsc.parallel_loop(0, K, unroll=2)
                def edge(e):
                    ad = abx[e]
                    if multi:
                        a_s, _ = plsc.unpack(
                            hbx[e, pl.ds(128, 32)],
                            format=plsc.PackFormat.INTERLEAVED)
                    else:
                        h2v, a_s = plsc.unpack(
                            hbx[e], format=plsc.PackFormat.INTERLEAVED)
                    ev = a_s + ad
                    ev = jnp.where(ev >= 0.0, ev, 0.2 * ev)
                    w = jnp.exp(ev)
                    cb[e, pl.ds(hw, 16)] = w
                    if multi:
                        for gg in range(4):
                            idx = jnp.int32(2 * gg) + (
                                jax.lax.iota(jnp.int32, 16) // 8)
                            wpair = w.at[idx].get(mode="promise_in_bounds")
                            ve, vo = plsc.unpack(
                                hbx[e, pl.ds(gg * 32, 32)],
                                format=plsc.PackFormat.INTERLEAVED)
                            cb[e, pl.ds(gg * 32, 16)] = ve * wpair
                            cb[e, pl.ds(gg * 32 + 16, 16)] = vo * wpair
                    else:
                        idx0 = jnp.zeros((16,), jnp.int32)
                        wb = w.at[idx0].get(mode="promise_in_bounds")
                        cb[e, pl.ds(0, 16)] = wb * h2v

                pltpu.async_copy(cb, acc.at[sds[u].at[1]], sss[b], add=True)
            return carry

        lax.fori_loop(0, CH // 4, quad, 0)
        pltpu.make_async_copy(cb, acc.at[pl.ds(0, K)],
                              sss[(CH - 1) % 2]).wait()
        plsc.subcore_barrier()
        pltpu.sync_copy(acc.at[pl.ds(rows0, NR)],
                        out_h.at[c, pl.ds(rows0, NR)])

    return edge_pass


_edge_pass_l1 = _make_edge_pass(H1, H1 * HID)
_edge_pass_l2 = _make_edge_pass(1, HID)


def kernel(x, edge_index, W1, a_src1, a_dst1, b1, W2, a_src2, a_dst2, b2,
           fc_w, fc_b):
    f32 = jnp.float32
    xp = jnp.zeros((NP, D), f32).at[:N].set(x)
    loop = jnp.arange(N, dtype=jnp.int32)
    pad = DUMMY + (jnp.arange(EP - ET, dtype=jnp.int32) % (NP - N))
    src = jnp.concatenate([edge_index[0], loop, pad])
    dst = jnp.concatenate([edge_index[1], loop, pad])
    sd = jnp.stack([src.reshape(NTILES * CH, K),
                    dst.reshape(NTILES * CH, K)], axis=1)

    cc = jnp.arange(D)
    m = (cc % 32) // 2
    P = 16 * (2 * (cc // 32) + m // 8) + 2 * (m % 8) + (cc % 2)
    Q = P[32 * (cc // 32) + 2 * (cc % 16) + (cc % 32) // 16]

    eye = jnp.eye(H1, 16, dtype=f32)
    As1 = (a_src1[:, :, None] * eye[:, None, :]).reshape(D, 16)
    Ad1 = (a_dst1[:, :, None] * eye[:, None, :]).reshape(D, 16)
    W1p = W1[:, P]
    C1 = jnp.zeros((D, 160), f32).at[:, :128].set(W1p)
    C1 = C1.at[:, 128::2].set(W1 @ As1)
    G1d = W1 @ Ad1
    RQ = (Q[None, :] // 16 == jnp.arange(16)[:, None]).astype(f32)
    b1Q = b1[Q].reshape(1, D)
    W2Q = W2[Q]
    av2 = W2Q @ a_src2[0]
    M2 = jnp.zeros((D, 32), f32).at[:, 0::2].set(W2Q).at[:, 1].set(av2)
    Ad2 = jnp.pad(a_dst2.T, ((0, 0), (0, 15)))
    NAd2 = W2Q @ Ad2

    sh = jax.ShapeDtypeStruct
    hbf, ad1 = _tc_call(
        _stage_a, [sh((NP, 160), jnp.bfloat16), sh((NP, 16), f32)])(
            xp, C1, G1d)
    acc1 = _edge_pass_l1(sd, hbf, ad1)
    hx2, ad2v = _tc_call(
        _stage_c, [sh((NP, 32), jnp.bfloat16), sh((NP, 16), f32)])(
            acc1, b1Q, RQ, M2, NAd2)
    acc2 = _edge_pass_l2(sd, hx2, ad2v)
    out = _tc_call(_stage_e, sh((N, 1), f32))(
        acc2[:, :N], b2.reshape(1, HID), fc_w, fc_b.reshape(1, 1))
    return out


# speedup vs baseline: 145.9328x; 1.0753x over previous
import functools

import jax
import jax.numpy as jnp
from jax import lax
from jax.experimental import pallas as pl
from jax.experimental.pallas import tpu as pltpu
from jax.experimental.pallas import tpu_sc as plsc

N = 10000
D = 128
HID = 16
H1 = 8

NP = 10112
DUMMY = N
E = 320000
ET = E + N
NTILES = 32
PER_TILE = 10752
EP = NTILES * PER_TILE
NR = NP // 16
K, CH = 112, 96


def _tc_call(f, out_shapes):
    return pl.pallas_call(f, out_shape=out_shapes)


def _stage_a(x_ref, c1_ref, g1d_ref, hbf_out, d_out):
    x = x_ref[...]
    h = jnp.dot(x, c1_ref[...], preferred_element_type=jnp.float32)
    hbf_out[...] = h.astype(jnp.bfloat16)
    d_out[...] = jnp.dot(x, g1d_ref[...], preferred_element_type=jnp.float32)


def _stage_c(acc_ref, b1_ref, rq_ref, m2_ref, nad_ref, hx2_out, d2_out):
    a0 = acc_ref[0]
    a1 = acc_ref[1]
    num = a0[:, :128] + a1[:, :128]
    sw = a0[:, 128:] + a1[:, 128:]
    sden = jnp.dot(sw, rq_ref[...], preferred_element_type=jnp.float32)
    h1 = num / sden + b1_ref[...]
    h1 = jnp.where(h1 > 0, h1, jnp.exp(h1) - 1.0)
    hx2_out[...] = jnp.dot(h1, m2_ref[...],
                           preferred_element_type=jnp.float32
                           ).astype(jnp.bfloat16)
    d2_out[...] = jnp.dot(h1, nad_ref[...], preferred_element_type=jnp.float32)


def _stage_e(acc_ref, b2_ref, fcw_ref, fcb_ref, out_ref):
    a0 = acc_ref[0]
    a1 = acc_ref[1]
    num = a0[:, :16] + a1[:, :16]
    s = a0[:, 16:17] + a1[:, 16:17]
    h = num / s + b2_ref[...]
    h = jnp.where(h > 0, h, jnp.exp(h) - 1.0)
    y = jnp.dot(h, fcw_ref[...], preferred_element_type=jnp.float32)
    y = y + fcb_ref[...]
    m = jnp.max(y, axis=0, keepdims=True)
    p = jnp.exp(y - m)
    out_ref[...] = p / jnp.sum(p, axis=0, keepdims=True)


def _make_edge_pass(heads, hw):
    accw = hw + 16
    multi = heads > 1
    hbw = 160 if multi else 32
    mesh = plsc.VectorSubcoreMesh(core_axis_name="c", subcore_axis_name="s")

    scratch = [
        pltpu.VMEM_SHARED((NP, accw), jnp.float32),
        pltpu.VMEM((2, K), jnp.int32),
        pltpu.VMEM((2, K), jnp.int32),
        pltpu.VMEM((2, K), jnp.int32),
        pltpu.VMEM((2, K), jnp.int32),
        pltpu.VMEM((K, hbw), jnp.bfloat16),
        pltpu.VMEM((K, hbw), jnp.bfloat16),
        pltpu.VMEM((K, 16), jnp.float32),
        pltpu.VMEM((K, 16), jnp.float32),
        pltpu.VMEM((K, accw), jnp.float32),
        pltpu.SemaphoreType.DMA,
        pltpu.SemaphoreType.DMA,
        pltpu.SemaphoreType.DMA,
        pltpu.SemaphoreType.DMA,
        pltpu.SemaphoreType.DMA,
        pltpu.SemaphoreType.DMA,
    ]

    @functools.partial(
        pl.kernel,
        mesh=mesh,
        compiler_params=pltpu.CompilerParams(needs_layout_passes=False,
                                             use_tc_tiling_on_sc=False),
        out_type=jax.ShapeDtypeStruct((2, NP, accw), jnp.float32),
        scratch_types=scratch,
    )
    def edge_pass(*refs):
        it = iter(refs)
        sd_h = next(it)
        hx_h = next(it)
        ad_h = next(it)
        out_h = next(it)
        acc = next(it)
        sds = (next(it), next(it), next(it), next(it))
        hbs = (next(it), next(it))
        abs_ = (next(it), next(it))
        cb = next(it)
        gss = (next(it), next(it))
        sss = (next(it), next(it))
        iss = (next(it), next(it))

        c = lax.axis_index("c")
        s = lax.axis_index("s")
        tile = c * 16 + s
        rows0 = s * NR
        cbase = tile * CH

        def issue_gather(r, bb):
            pltpu.async_copy(hx_h.at[sds[r].at[0]], hbs[bb], gss[bb])
            pltpu.async_copy(ad_h.at[sds[r].at[1]], abs_[bb], gss[bb])

        def wait_gather(bb):
            pltpu.make_async_copy(hx_h.at[pl.ds(0, K)], hbs[bb],
                                  gss[bb]).wait()
            pltpu.make_async_copy(ad_h.at[pl.ds(0, K)], abs_[bb],
                                  gss[bb]).wait()

        pltpu.sync_copy(sd_h.at[cbase], sds[0])
        pltpu.sync_copy(sd_h.at[cbase + 1], sds[1])
        issue_gather(0, 0)

        @plsc.parallel_loop(0, K, unroll=2)
        def zrow(r):
            for j in range(accw // 16):
                cb[r, pl.ds(j * 16, 16)] = jnp.zeros((16,), jnp.float32)

        for j in range(NR // K):
            pltpu.sync_copy(cb, acc.at[pl.ds(rows0 + j * K, K)])
        if NR % K:
            pltpu.sync_copy(cb.at[pl.ds(0, NR % K)],
                            acc.at[pl.ds(rows0 + (NR // K) * K, NR % K)])
        plsc.subcore_barrier()

        def quad(g, carry):
            for u in range(4):
                b = u % 2
                cur = 4 * g + u
                hbx = hbs[b]
                abx = abs_[b]

                @pl.when(cur + 1 < CH)
                def _():
                    @pl.when(cur >= 1)
                    def _():
                        pltpu.make_async_copy(sd_h.at[cbase],
                                              sds[(u + 1) % 4],
                                              iss[1 - b]).wait()
                    issue_gather((u + 1) % 4, 1 - b)

                @pl.when(cur + 2 < CH)
                def _():
                    pltpu.async_copy(sd_h.at[cbase + cur + 2],
                                     sds[(u + 2) % 4], iss[b])

                wait_gather(b)

                @pl.when(cur >= 1)
                def _():
                    pltpu.make_async_copy(cb, acc.at[pl.ds(0, K)],
                                          sss[1 - b]).wait()

                @plsc.parallel_loop(0, K, unroll=2)
                def edge(e):
                    ad = abx[e]
                    if multi:
                        a_s, _ = plsc.unpack(
                            hbx[e, pl.ds(128, 32)],
                            format=plsc.PackFormat.INTERLEAVED)
                    else:
                        h2v, a_s = plsc.unpack(
                            hbx[e], format=plsc.PackFormat.INTERLEAVED)
                    ev = a_s + ad
                    ev = jnp.where(ev >= 0.0, ev, 0.2 * ev)
                    w = jnp.exp(ev)
                    cb[e, pl.ds(hw, 16)] = w
                    if multi:
                        for gg in range(4):
                            idx = jnp.int32(2 * gg) + (
                                jax.lax.iota(jnp.int32, 16) // 8)
                            wpair = w.at[idx].get(mode="promise_in_bounds")
                            ve, vo = plsc.unpack(
                                hbx[e, pl.ds(gg * 32, 32)],
                                format=plsc.PackFormat.INTERLEAVED)
                            cb[e, pl.ds(gg * 32, 16)] = ve * wpair
                            cb[e, pl.ds(gg * 32 + 16, 16)] = vo * wpair
                    else:
                        idx0 = jnp.zeros((16,), jnp.int32)
                        wb = w.at[idx0].get(mode="promise_in_bounds")
                        cb[e, pl.ds(0, 16)] = wb * h2v

                pltpu.async_copy(cb, acc.at[sds[u].at[1]], sss[b], add=True)
            return carry

        lax.fori_loop(0, CH // 4, quad, 0)
        pltpu.make_async_copy(cb, acc.at[pl.ds(0, K)],
                              sss[(CH - 1) % 2]).wait()
        plsc.subcore_barrier()
        pltpu.sync_copy(acc.at[pl.ds(rows0, NR)],
                        out_h.at[c, pl.ds(rows0, NR)])

    return edge_pass


_edge_pass_l1 = _make_edge_pass(H1, H1 * HID)
_edge_pass_l2 = _make_edge_pass(1, HID)


def kernel(x, edge_index, W1, a_src1, a_dst1, b1, W2, a_src2, a_dst2, b2,
           fc_w, fc_b):
    f32 = jnp.float32
    xp = jnp.zeros((NP, D), f32).at[:N].set(x)
    loop = jnp.arange(N, dtype=jnp.int32)
    pad = DUMMY + (jnp.arange(EP - ET, dtype=jnp.int32) % (NP - N))
    src = jnp.concatenate([edge_index[0], loop, pad])
    dst = jnp.concatenate([edge_index[1], loop, pad])
    sd = jnp.stack([src.reshape(NTILES * CH, K),
                    dst.reshape(NTILES * CH, K)], axis=1)

    cc = jnp.arange(D)
    m = (cc % 32) // 2
    P = 16 * (2 * (cc // 32) + m // 8) + 2 * (m % 8) + (cc % 2)
    Q = P[32 * (cc // 32) + 2 * (cc % 16) + (cc % 32) // 16]

    eye = jnp.eye(H1, 16, dtype=f32)
    As1 = (a_src1[:, :, None] * eye[:, None, :]).reshape(D, 16)
    Ad1 = (a_dst1[:, :, None] * eye[:, None, :]).reshape(D, 16)
    W1p = W1[:, P]
    C1 = jnp.zeros((D, 160), f32).at[:, :128].set(W1p)
    C1 = C1.at[:, 128::2].set(W1 @ As1)
    G1d = W1 @ Ad1
    RQ = (Q[None, :] // 16 == jnp.arange(16)[:, None]).astype(f32)
    b1Q = b1[Q].reshape(1, D)
    W2Q = W2[Q]
    av2 = W2Q @ a_src2[0]
    M2 = jnp.zeros((D, 32), f32).at[:, 0::2].set(W2Q).at[:, 1].set(av2)
    Ad2 = jnp.pad(a_dst2.T, ((0, 0), (0, 15)))
    NAd2 = W2Q @ Ad2

    sh = jax.ShapeDtypeStruct
    hbf, ad1 = _tc_call(
        _stage_a, [sh((NP, 160), jnp.bfloat16), sh((NP, 16), f32)])(
            xp, C1, G1d)
    acc1 = _edge_pass_l1(sd, hbf, ad1)
    hx2, ad2v = _tc_call(
        _stage_c, [sh((NP, 32), jnp.bfloat16), sh((NP, 16), f32)])(
            acc1, b1Q, RQ, M2, NAd2)
    acc2 = _edge_pass_l2(sd, hx2, ad2v)
    out = _tc_call(_stage_e, sh((N, 1), f32))(
        acc2[:, :N], b2.reshape(1, HID), fc_w, fc_b.reshape(1, 1))
    return out
